# baseline (speedup 1.0000x reference)
import jax
import jax.numpy as jnp
from jax.experimental import pallas as pl
from jax.experimental.pallas import tpu as pltpu

N = 10000
E = 160000
D_IN = 128
H = 8
D = 128
G = 100


def _gat_conv(x, src, dst, W, al, ar, Wr):
    feat = (x @ W).reshape(N, H, D)
    el = jnp.sum(feat * al[None, :, :], axis=-1)
    er = jnp.sum(feat * ar[None, :, :], axis=-1)
    e = jax.nn.leaky_relu(el[src] + er[dst], negative_slope=0.2)
    emax = jax.ops.segment_max(e, dst, num_segments=N)
    ee = jnp.exp(e - emax[dst])
    denom = jax.ops.segment_sum(ee, dst, num_segments=N)
    alpha = ee / (denom[dst] + 1e-9)
    heads = []
    for h in range(H):
        msg = feat[:, h, :][src] * alpha[:, h:h + 1]
        heads.append(jax.ops.segment_sum(msg, dst, num_segments=N))
    out = jnp.stack(heads, axis=1)
    res = (x @ Wr).reshape(N, H, D)
    return out + res


def _combine_kernel(h1_ref, h2_ref, mean_ref, out_ref):
    s = h1_ref[...] + h2_ref[...]
    out_ref[...] = jnp.max(s, axis=1) + mean_ref[...]


def kernel(x, edge_index, graph_ids, W1, al1, ar1, Wr1, W2, al2, ar2, Wr2):
    src = edge_index[0]
    dst = edge_index[1]
    h1 = _gat_conv(x, src, dst, W1, al1, ar1, Wr1)
    h2 = _gat_conv(x, dst, src, W2, al2, ar2, Wr2)
    sums = jax.ops.segment_sum(x, graph_ids, num_segments=G)
    cnts = jax.ops.segment_sum(jnp.ones((N, 1), dtype=x.dtype), graph_ids, num_segments=G)
    mean = sums / jnp.maximum(cnts, 1.0)
    meanfull = mean[graph_ids]

    bn = 1000
    out = pl.pallas_call(
        _combine_kernel,
        grid=(N // bn,),
        in_specs=[
            pl.BlockSpec((bn, H, D), lambda i: (i, 0, 0)),
            pl.BlockSpec((bn, H, D), lambda i: (i, 0, 0)),
            pl.BlockSpec((bn, D), lambda i: (i, 0)),
        ],
        out_specs=pl.BlockSpec((bn, D), lambda i: (i, 0)),
        out_shape=jax.ShapeDtypeStruct((N, D), jnp.float32),
    )(h1, h2, meanfull)
    return out


# speedup vs baseline: 3.6326x; 3.6326x over previous
import functools

import jax
import jax.numpy as jnp
from jax import lax
from jax.experimental import pallas as pl
from jax.experimental.pallas import tpu as pltpu
from jax.experimental.pallas import tpu_sc as plsc

N = 10000
E = 160000
D_IN = 128
H = 8
D = 128
G = 100
GP = 104
BN = 1000
NEG_SLOPE = 0.2


def _leaky(v):
    return jnp.maximum(v, NEG_SLOPE * v)


def _stage_a_body(x_ref, w1_ref, w2_ref, wr_ref, al1_ref, ar1_ref, al2_ref,
                  ar2_ref, gids_ref,
                  f1_ref, f2_ref, res_ref, elrs_ref, elrd_ref, cb_ref, mg_ref,
                  max_sc, sums_sc, cnts_sc):
    i = pl.program_id(0)

    @pl.when(i == 0)
    def _init():
        max_sc[...] = jnp.full((8, 16), -jnp.inf, jnp.float32)
        sums_sc[...] = jnp.zeros((GP, D), jnp.float32)
        cnts_sc[...] = jnp.zeros((GP, D), jnp.float32)

    xb = x_ref[...]
    t1 = jnp.dot(xb, w1_ref[...], preferred_element_type=jnp.float32)
    t2 = jnp.dot(xb, w2_ref[...], preferred_element_type=jnp.float32)
    res = jnp.dot(xb, wr_ref[...], preferred_element_type=jnp.float32)
    for h in range(H):
        f1_ref[h] = t1[:, h * D:(h + 1) * D]
        f2_ref[h] = t2[:, h * D:(h + 1) * D]
        res_ref[h] = res[:, h * D:(h + 1) * D]
    t13 = t1.reshape(BN, H, D)
    t23 = t2.reshape(BN, H, D)
    el1 = jnp.sum(t13 * al1_ref[...][None, :, :], axis=2)
    er1 = jnp.sum(t13 * ar1_ref[...][None, :, :], axis=2)
    el2 = jnp.sum(t23 * al2_ref[...][None, :, :], axis=2)
    er2 = jnp.sum(t23 * ar2_ref[...][None, :, :], axis=2)
    es = jnp.concatenate([el1, er2], axis=1)
    ed = jnp.concatenate([er1, el2], axis=1)
    zpad = jnp.zeros((BN, 112), jnp.float32)
    elrs_ref[...] = jnp.concatenate([es, zpad], axis=1)
    elrd_ref[...] = jnp.concatenate([ed, zpad], axis=1)
    max_sc[0] = jnp.maximum(max_sc[0], jnp.max(es, axis=0))
    max_sc[1] = jnp.maximum(max_sc[1], jnp.max(ed, axis=0))

    gids = gids_ref[0]
    mh = (lax.broadcasted_iota(jnp.int32, (GP, BN), 0)
          == jnp.broadcast_to(gids, (GP, BN))).astype(jnp.float32)
    sums_sc[...] += jnp.dot(mh, xb, preferred_element_type=jnp.float32)
    cnts_sc[...] += jnp.broadcast_to(
        jnp.sum(mh, axis=1, keepdims=True), (GP, D))

    @pl.when(i == pl.num_programs(0) - 1)
    def _fin():
        m = max_sc[...]
        cb_ref[...] = _leaky(m[0:1, :] + m[1:2, :])
        mg_ref[...] = sums_sc[...] / jnp.maximum(cnts_sc[...], 1.0)


def _stage_a(x, gids3, W1, W2, Wr, al1, ar1, al2, ar2):
    grid = (N // BN,)
    out_shapes = (
        jax.ShapeDtypeStruct((H, N, D), jnp.float32),
        jax.ShapeDtypeStruct((H, N, D), jnp.float32),
        jax.ShapeDtypeStruct((H, N, D), jnp.float32),
        jax.ShapeDtypeStruct((N, 128), jnp.float32),
        jax.ShapeDtypeStruct((N, 128), jnp.float32),
        jax.ShapeDtypeStruct((1, 16), jnp.float32),
        jax.ShapeDtypeStruct((GP, D), jnp.float32),
    )
    return pl.pallas_call(
        _stage_a_body,
        grid=grid,
        in_specs=[
            pl.BlockSpec((BN, D_IN), lambda i: (i, 0)),
            pl.BlockSpec((D_IN, H * D), lambda i: (0, 0)),
            pl.BlockSpec((D_IN, H * D), lambda i: (0, 0)),
            pl.BlockSpec((D_IN, H * D), lambda i: (0, 0)),
            pl.BlockSpec((H, D), lambda i: (0, 0)),
            pl.BlockSpec((H, D), lambda i: (0, 0)),
            pl.BlockSpec((H, D), lambda i: (0, 0)),
            pl.BlockSpec((H, D), lambda i: (0, 0)),
            pl.BlockSpec((1, 1, BN), lambda i: (i, 0, 0)),
        ],
        out_specs=[
            pl.BlockSpec((H, BN, D), lambda i: (0, i, 0)),
            pl.BlockSpec((H, BN, D), lambda i: (0, i, 0)),
            pl.BlockSpec((H, BN, D), lambda i: (0, i, 0)),
            pl.BlockSpec((BN, 128), lambda i: (i, 0)),
            pl.BlockSpec((BN, 128), lambda i: (i, 0)),
            pl.BlockSpec((1, 16), lambda i: (0, 0)),
            pl.BlockSpec((GP, D), lambda i: (0, 0)),
        ],
        out_shape=out_shapes,
        scratch_shapes=[
            pltpu.VMEM((8, 16), jnp.float32),
            pltpu.VMEM((GP, D), jnp.float32),
            pltpu.VMEM((GP, D), jnp.float32),
        ],
        compiler_params=pltpu.CompilerParams(
            dimension_semantics=("arbitrary",)),
    )(x, W1, W2, Wr, al1, ar1, al2, ar2, gids3)


def _stage_d_body(a1_ref, a2_ref, res_ref, dA_ref, dB_ref, gids_ref, mg_ref,
                  out_ref):
    rd1 = 1.0 / (dA_ref[...][:, 0:8] + 1e-9)
    rd2 = 1.0 / (dB_ref[...][:, 8:16] + 1e-9)
    m = None
    for h in range(H):
        v = (a1_ref[h] * rd1[:, h:h + 1] + a2_ref[h] * rd2[:, h:h + 1]
             + res_ref[h])
        m = v if m is None else jnp.maximum(m, v)
    gids = gids_ref[0]
    mh = (lax.broadcasted_iota(jnp.int32, (GP, BN), 0)
          == jnp.broadcast_to(gids, (GP, BN))).astype(jnp.float32)
    mf = lax.dot_general(mh, mg_ref[...], (((0,), (0,)), ((), ())),
                         preferred_element_type=jnp.float32)
    out_ref[...] = m + mf


def _stage_d(acc1, acc2, resh, denomA, denomB, gids3, mean_g):
    return pl.pallas_call(
        _stage_d_body,
        grid=(N // BN,),
        in_specs=[
            pl.BlockSpec((H, BN, D), lambda i: (0, i, 0)),
            pl.BlockSpec((H, BN, D), lambda i: (0, i, 0)),
            pl.BlockSpec((H, BN, D), lambda i: (0, i, 0)),
            pl.BlockSpec((BN, D), lambda i: (i, 0)),
            pl.BlockSpec((BN, D), lambda i: (i, 0)),
            pl.BlockSpec((1, 1, BN), lambda i: (i, 0, 0)),
            pl.BlockSpec((GP, D), lambda i: (0, 0)),
        ],
        out_specs=pl.BlockSpec((BN, D), lambda i: (i, 0)),
        out_shape=jax.ShapeDtypeStruct((N, D), jnp.float32),
        compiler_params=pltpu.CompilerParams(
            dimension_semantics=("arbitrary",)),
    )(acc1, acc2, resh, denomA, denomB, gids3, mean_g)


_MESH = plsc.VectorSubcoreMesh(core_axis_name="c", subcore_axis_name="s")
NT = 16
EPT = E // NT
CH = 125
NCH = EPT // CH
NPT = N // NT


GRP = 1000
NG = EPT // GRP
NPT0 = 624


def _zero_rows(buf, nrows, ncols):
    zz = jnp.zeros((16,), jnp.float32)

    def _z(r, z):
        for j in range(ncols // 16):
            buf[r, pl.ds(j * 16, 16)] = zz
        return z
    lax.fori_loop(0, nrows, _z, 0)


CB = 50
RPTB = EPT // CB
NGB = RPTB // 8


def _acc_slices(sid):
    base = pl.multiple_of(sid * NPT0, 8)
    hops = [(base + 80 * o, 80) for o in range(7)] + [(base + 560, 64)]
    return hops, (15 * NPT0 + 624, 16)


def _acc_zero(dacc, bounce, sid):
    hops, (x0, xs) = _acc_slices(sid)
    for off, sz in hops:
        pltpu.sync_copy(bounce.at[pl.ds(0, sz)], dacc.at[pl.ds(off, sz)])

    @pl.when(sid == 15)
    def _():
        pltpu.sync_copy(bounce.at[pl.ds(0, xs)], dacc.at[pl.ds(x0, xs)])


def _acc_export(dacc, out, bounce, sid, obase):
    hops, (x0, xs) = _acc_slices(sid)
    for off, sz in hops:
        pltpu.sync_copy(dacc.at[pl.ds(off, sz)], bounce.at[pl.ds(0, sz)])
        oo = pl.multiple_of(obase + off, 8)
        pltpu.sync_copy(bounce.at[pl.ds(0, sz)], out.at[pl.ds(oo, sz)])

    @pl.when(sid == 15)
    def _():
        pltpu.sync_copy(dacc.at[pl.ds(x0, xs)], bounce.at[pl.ds(0, xs)])
        oo = pl.multiple_of(obase + x0, 8)
        pltpu.sync_copy(bounce.at[pl.ds(0, xs)], out.at[pl.ds(oo, xs)])


def _stage_b_body(elrs_hbm, elrd_hbm, eidx_hbm, cb_hbm,
                  ee_out, dA_out, dB_out,
                  srcg, dstg, rowsA, eec, eev, cbv, bounce, dacc, sem):
    cid = lax.axis_index("c")
    sid = lax.axis_index("s")

    _zero_rows(bounce, 80, D)
    _acc_zero(dacc, bounce, sid)
    plsc.subcore_barrier()

    pltpu.sync_copy(cb_hbm, cbv)
    cb16 = cbv[0]

    def _group(g, z):
        row0 = pl.multiple_of(sid * RPTB + g * 8, 8)
        pltpu.sync_copy(eidx_hbm.at[0, pl.ds(row0, 8)], srcg)
        pltpu.sync_copy(eidx_hbm.at[1, pl.ds(row0, 8)], dstg)
        for c in range(8):
            pltpu.async_copy(elrs_hbm.at[srcg.at[c]], rowsA, sem).wait()
            pltpu.async_copy(elrd_hbm.at[dstg.at[c]], eec, sem).wait()

            def _edge(i, zz_):
                a = rowsA[i, pl.ds(0, 16)]
                b = eec[i, pl.ds(0, 16)]
                s = a + b
                v = jnp.exp(jnp.maximum(s, NEG_SLOPE * s) - cb16)
                eec[i, pl.ds(0, 16)] = v
                eev[i] = v
                return zz_
            lax.fori_loop(0, CB, _edge, 0)

            @pl.when(cid == 0)
            def _():
                pltpu.sync_copy(eec, dacc.at[dstg.at[c]], add=True)

            @pl.when(cid == 1)
            def _():
                pltpu.sync_copy(eec, dacc.at[srcg.at[c]], add=True)

            @pl.when(cid == jnp.where(g < 13, 0, 1))
            def _():
                pltpu.sync_copy(eev, ee_out.at[row0 + c])
        return z

    lax.fori_loop(0, NGB, _group, 0)
    plsc.subcore_barrier()

    @pl.when(cid == 0)
    def _():
        _acc_export(dacc, dA_out, bounce, sid, 0)

    @pl.when(cid == 1)
    def _():
        _acc_export(dacc, dB_out, bounce, sid, 0)


def _stage_b(elrs, elrd, eidx3b, cb):
    f = pl.kernel(
        _stage_b_body,
        out_type=(
            jax.ShapeDtypeStruct((E // CB, CB, 16), jnp.float32),
            jax.ShapeDtypeStruct((N, D), jnp.float32),
            jax.ShapeDtypeStruct((N, D), jnp.float32),
        ),
        mesh=_MESH,
        scratch_types=[
            pltpu.VMEM((8, CB), jnp.int32),
            pltpu.VMEM((8, CB), jnp.int32),
            pltpu.VMEM((CB, 128), jnp.float32),
            pltpu.VMEM((CB, 128), jnp.float32),
            pltpu.VMEM((CB, 16), jnp.float32),
            pltpu.VMEM((1, 16), jnp.float32),
            pltpu.VMEM((80, D), jnp.float32),
            pltpu.VMEM_SHARED((N, D), jnp.float32),
            pltpu.SemaphoreType.DMA,
        ],
        compiler_params=pltpu.CompilerParams(needs_layout_passes=False),
    )
    return f(elrs, elrd, eidx3b, cb)


def _stage_c_body(f1, f2, is8, id8, ei3, eet,
                  o1, o2,
                  gidx, sidx, eeg, rows, zbuf, acc, sem):
    cid = lax.axis_index("c")
    sid = lax.axis_index("s")
    _zero_rows(zbuf, 80, D)

    def _head(h, z):
        _acc_zero(acc, zbuf, sid)
        plsc.subcore_barrier()

        def _group(g, zz):
            row0 = pl.multiple_of(sid * (EPT // CH) + g * 8, 8)

            @pl.when(cid == 0)
            def _():
                pltpu.sync_copy(is8.at[h, pl.ds(row0, 8)], gidx)
                pltpu.sync_copy(ei3.at[1, pl.ds(row0, 8)], sidx)
                pltpu.sync_copy(eet.at[h, pl.ds(row0, 8)], eeg)

            @pl.when(cid == 1)
            def _():
                pltpu.sync_copy(id8.at[h, pl.ds(row0, 8)], gidx)
                pltpu.sync_copy(ei3.at[0, pl.ds(row0, 8)], sidx)
                pltpu.sync_copy(eet.at[h + 8, pl.ds(row0, 8)], eeg)

            for c in range(8):
                @pl.when(cid == 0)
                def _():
                    pltpu.async_copy(f1.at[gidx.at[c]], rows, sem).wait()

                @pl.when(cid == 1)
                def _():
                    pltpu.async_copy(f2.at[gidx.at[c]], rows, sem).wait()

                def _edge(i, zzz):
                    sp = plsc.load_gather(
                        eeg, [jnp.full((16,), c, jnp.int32),
                              jnp.full((16,), i, jnp.int32)])
                    for j in range(8):
                        sl = pl.ds(j * 16, 16)
                        rows[i, sl] = rows[i, sl] * sp
                    return zzz
                lax.fori_loop(0, CH, _edge, 0)
                pltpu.sync_copy(rows, acc.at[sidx.at[c]], add=True)
            return zz

        lax.fori_loop(0, NG, _group, 0)
        plsc.subcore_barrier()

        @pl.when(cid == 0)
        def _():
            _acc_export(acc, o1, rows, sid, h * N)

        @pl.when(cid == 1)
        def _():
            _acc_export(acc, o2, rows, sid, h * N)
        return z

    lax.fori_loop(0, H, _head, 0)


def _stage_c(feat1f, feat2f, idxsrc8, idxdst8, eidx3, eeT):
    f = pl.kernel(
        _stage_c_body,
        out_type=(
            jax.ShapeDtypeStruct((H * N, D), jnp.float32),
            jax.ShapeDtypeStruct((H * N, D), jnp.float32),
        ),
        mesh=_MESH,
        scratch_types=[
            pltpu.VMEM((8, CH), jnp.int32),
            pltpu.VMEM((8, CH), jnp.int32),
            pltpu.VMEM((8, CH), jnp.float32),
            pltpu.VMEM((CH, D), jnp.float32),
            pltpu.VMEM((80, D), jnp.float32),
            pltpu.VMEM_SHARED((N, D), jnp.float32),
            pltpu.SemaphoreType.DMA,
        ],
        compiler_params=pltpu.CompilerParams(needs_layout_passes=False),
    )
    return f(feat1f, feat2f, idxsrc8, idxdst8, eidx3, eeT)


def kernel(x, edge_index, graph_ids, W1, al1, ar1, Wr1, W2, al2, ar2, Wr2):
    src = edge_index[0]
    dst = edge_index[1]
    gids3 = graph_ids.reshape(N // BN, 1, BN)
    Wr = Wr1 + Wr2

    (feat1h, feat2h, resh, elr_src, elr_dst, cb, mean_g) = _stage_a(
        x, gids3, W1, W2, Wr, al1, ar1, al2, ar2)

    eidx3b = edge_index.reshape(2, E // CB, CB)
    ee3, denomA, denomB = _stage_b(elr_src, elr_dst, eidx3b, cb)

    eidx3 = edge_index.reshape(2, E // CH, CH)
    offs = (jnp.arange(H, dtype=jnp.int32) * N)[:, None]
    idxsrc8 = (src[None, :] + offs).reshape(H, E // CH, CH)
    idxdst8 = (dst[None, :] + offs).reshape(H, E // CH, CH)
    eeT = ee3.reshape(E, 16).T.reshape(16, E // CH, CH)
    out1, out2 = _stage_c(feat1h.reshape(H * N, D), feat2h.reshape(H * N, D),
                          idxsrc8, idxdst8, eidx3, eeT)
    acc1 = out1.reshape(H, N, D)
    acc2 = out2.reshape(H, N, D)

    return _stage_d(acc1, acc2, resh, denomA, denomB, gids3, mean_g)
